# baseline (speedup 1.0000x reference)
import functools

import jax
import jax.numpy as jnp
from jax import lax
from jax.experimental import pallas as pl
from jax.experimental.pallas import tpu as pltpu
from jax.experimental.pallas import tpu_sc as plsc

NUM_CORES = 2
NUM_SUBCORES = 16
NUM_WORKERS = NUM_CORES * NUM_SUBCORES
LANES = 16
BATCH_N = 16384
FEAT = 64
NUM_ROWS = 1000000
ROWS_PER_W = BATCH_N // NUM_WORKERS
CHUNK = 128
NCHUNK = ROWS_PER_W // CHUNK


def _body(user_hbm, item_hbm, ut_hbm, it_hbm, out_hbm,
          uidx_v, iidx_v, us_v, is_v, out_v, sem):
    wid = lax.axis_index("s") * NUM_CORES + lax.axis_index("c")
    base = wid * ROWS_PER_W

    pltpu.sync_copy(user_hbm.at[pl.ds(base, ROWS_PER_W)], uidx_v)
    pltpu.sync_copy(item_hbm.at[pl.ds(base, ROWS_PER_W)], iidx_v)

    for j in range(NCHUNK):
        usl = uidx_v.at[pl.ds(j * CHUNK, CHUNK)]
        isl = iidx_v.at[pl.ds(j * CHUNK, CHUNK)]

        def fire(c, _):
            pltpu.async_copy(ut_hbm.at[c].at[usl],
                             us_v.at[c, pl.ds(j * CHUNK, CHUNK)], sem)
            pltpu.async_copy(it_hbm.at[c].at[isl],
                             is_v.at[c, pl.ds(j * CHUNK, CHUNK)], sem)
            return ()

        lax.fori_loop(0, FEAT, fire, ())

        def drain(c, _):
            pltpu.make_async_copy(
                ut_hbm.at[0].at[pl.ds(0, CHUNK)],
                us_v.at[c, pl.ds(j * CHUNK, CHUNK)], sem).wait()
            pltpu.make_async_copy(
                it_hbm.at[0].at[pl.ds(0, CHUNK)],
                is_v.at[c, pl.ds(j * CHUNK, CHUNK)], sem).wait()
            return ()

        lax.fori_loop(0, FEAT, drain, ())

    def grp_body(g, _):
        sl = pl.ds(g * LANES, LANES)
        acc = jnp.zeros((LANES,), jnp.float32)
        accs = [jnp.zeros((LANES,), jnp.float32) for _ in range(4)]
        for c in range(FEAT):
            accs[c % 4] = accs[c % 4] + us_v[c, sl] * is_v[c, sl]
        out_v[sl] = (accs[0] + accs[1]) + (accs[2] + accs[3])
        return ()

    lax.fori_loop(0, ROWS_PER_W // LANES, grp_body, ())

    pltpu.sync_copy(out_v, out_hbm.at[pl.ds(base, ROWS_PER_W)])


@jax.jit
def kernel(user, item, users_emb, items_emb):
    ut = users_emb.T
    it = items_emb.T
    mesh = plsc.VectorSubcoreMesh(core_axis_name="c", subcore_axis_name="s")
    k = pl.kernel(
        _body,
        out_type=jax.ShapeDtypeStruct((BATCH_N,), jnp.float32),
        mesh=mesh,
        scratch_types=[
            pltpu.VMEM((ROWS_PER_W,), jnp.int32),
            pltpu.VMEM((ROWS_PER_W,), jnp.int32),
            pltpu.VMEM((FEAT, ROWS_PER_W), jnp.float32),
            pltpu.VMEM((FEAT, ROWS_PER_W), jnp.float32),
            pltpu.VMEM((ROWS_PER_W,), jnp.float32),
            pltpu.SemaphoreType.DMA,
        ],
        compiler_params=pltpu.CompilerParams(
            needs_layout_passes=False, use_tc_tiling_on_sc=False),
    )
    return k(user.astype(jnp.int32), item.astype(jnp.int32), ut, it)


# speedup vs baseline: 1.0950x; 1.0950x over previous
import functools

import jax
import jax.numpy as jnp
from jax import lax
from jax.experimental import pallas as pl
from jax.experimental.pallas import tpu as pltpu
from jax.experimental.pallas import tpu_sc as plsc

NUM_CORES = 2
NUM_SUBCORES = 16
NUM_WORKERS = NUM_CORES * NUM_SUBCORES
LANES = 16
BATCH_N = 16384
FEAT = 64
NUM_ROWS = 1000000
ROWS_PER_W = BATCH_N // NUM_WORKERS
CHUNK = 128
NCHUNK = ROWS_PER_W // CHUNK
ROW_PAD = 1000064


def _body(user_hbm, item_hbm, ut_hbm, it_hbm, out_hbm,
          uidx_v, iidx_v, us_v, is_v, out_v, sem):
    wid = lax.axis_index("s") * NUM_CORES + lax.axis_index("c")
    base = wid * ROWS_PER_W

    pltpu.sync_copy(user_hbm.at[pl.ds(base, ROWS_PER_W)], uidx_v)
    pltpu.sync_copy(item_hbm.at[pl.ds(base, ROWS_PER_W)], iidx_v)

    for j in range(NCHUNK):
        usl = uidx_v.at[pl.ds(j * CHUNK, CHUNK)]
        isl = iidx_v.at[pl.ds(j * CHUNK, CHUNK)]

        def fire(c, _):
            pltpu.async_copy(ut_hbm.at[c].at[usl],
                             us_v.at[c, pl.ds(j * CHUNK, CHUNK)], sem)
            pltpu.async_copy(it_hbm.at[c].at[isl],
                             is_v.at[c, pl.ds(j * CHUNK, CHUNK)], sem)
            return ()

        lax.fori_loop(0, FEAT, fire, ())

        def drain(c, _):
            pltpu.make_async_copy(
                ut_hbm.at[0].at[pl.ds(0, CHUNK)],
                us_v.at[c, pl.ds(j * CHUNK, CHUNK)], sem).wait()
            pltpu.make_async_copy(
                it_hbm.at[0].at[pl.ds(0, CHUNK)],
                is_v.at[c, pl.ds(j * CHUNK, CHUNK)], sem).wait()
            return ()

        lax.fori_loop(0, FEAT, drain, ())

    def grp_body(g, _):
        sl = pl.ds(g * LANES, LANES)
        accs = [jnp.zeros((LANES,), jnp.float32) for _ in range(4)]
        for c in range(FEAT):
            accs[c % 4] = accs[c % 4] + us_v[c, sl] * is_v[c, sl]
        out_v[sl] = (accs[0] + accs[1]) + (accs[2] + accs[3])
        return ()

    lax.fori_loop(0, ROWS_PER_W // LANES, grp_body, ())

    pltpu.sync_copy(out_v, out_hbm.at[pl.ds(base, ROWS_PER_W)])


DSUB = 13
DBLK = ROW_PAD // DSUB


def _depad_body(in_ref, o_ref):
    o_ref[...] = in_ref[...]


def _tc_depad(embt):
    return pl.pallas_call(
        _depad_body,
        out_shape=jax.ShapeDtypeStruct((FEAT, ROW_PAD), jnp.float32),
        grid=(FEAT // 8, DSUB),
        in_specs=[pl.BlockSpec((8, DBLK), lambda fc, rb: (fc, rb))],
        out_specs=pl.BlockSpec((8, DBLK), lambda fc, rb: (fc, rb)),
    )(embt)


@jax.jit
def kernel(user, item, users_emb, items_emb):
    uflat = _tc_depad(users_emb.T)
    iflat = _tc_depad(items_emb.T)
    mesh = plsc.VectorSubcoreMesh(core_axis_name="c", subcore_axis_name="s")
    k = pl.kernel(
        _body,
        out_type=jax.ShapeDtypeStruct((BATCH_N,), jnp.float32),
        mesh=mesh,
        scratch_types=[
            pltpu.VMEM((ROWS_PER_W,), jnp.int32),
            pltpu.VMEM((ROWS_PER_W,), jnp.int32),
            pltpu.VMEM((FEAT, ROWS_PER_W), jnp.float32),
            pltpu.VMEM((FEAT, ROWS_PER_W), jnp.float32),
            pltpu.VMEM((ROWS_PER_W,), jnp.float32),
            pltpu.SemaphoreType.DMA,
        ],
        compiler_params=pltpu.CompilerParams(
            needs_layout_passes=False, use_tc_tiling_on_sc=False),
    )
    return k(user.astype(jnp.int32), item.astype(jnp.int32), uflat, iflat)


# speedup vs baseline: 14.5213x; 13.2619x over previous
import functools

import jax
import jax.numpy as jnp
from jax import lax
from jax.experimental import pallas as pl
from jax.experimental.pallas import tpu as pltpu
from jax.experimental.pallas import tpu_sc as plsc

NUM_CORES = 2
NUM_SUBCORES = 16
NUM_WORKERS = NUM_CORES * NUM_SUBCORES
LANES = 16
BATCH_N = 16384
FEAT = 64
ROWS_PER_W = BATCH_N // NUM_WORKERS
CHUNK = 256
NCHUNK = ROWS_PER_W // CHUNK


def _body(user_hbm, item_hbm, uemb_hbm, iemb_hbm, out_hbm,
          uidx_v, iidx_v, urows_v, irows_v, out_v, sem):
    wid = lax.axis_index("s") * NUM_CORES + lax.axis_index("c")
    base = wid * ROWS_PER_W

    pltpu.sync_copy(user_hbm.at[pl.ds(base, ROWS_PER_W)], uidx_v)
    pltpu.sync_copy(item_hbm.at[pl.ds(base, ROWS_PER_W)], iidx_v)

    lane = lax.iota(jnp.int32, LANES)

    def chunk_body(c, _):
        lo = c * CHUNK

        def fire(g, _):
            uvec = uidx_v[pl.ds(lo + g * LANES, LANES)]
            ivec = iidx_v[pl.ds(lo + g * LANES, LANES)]
            for rr in range(LANES):
                k = g * LANES + rr
                pltpu.async_copy(uemb_hbm.at[uvec[rr]],
                                 urows_v.at[k, 0], sem)
                pltpu.async_copy(iemb_hbm.at[ivec[rr]],
                                 irows_v.at[k, 0], sem)
            return ()

        lax.fori_loop(0, CHUNK // LANES, fire, ())

        def drain(k, _):
            pltpu.make_async_copy(
                uemb_hbm.at[0], urows_v.at[k, 0], sem).wait()
            pltpu.make_async_copy(
                iemb_hbm.at[0], irows_v.at[k, 0], sem).wait()
            return ()

        lax.fori_loop(0, CHUNK, drain, ())

        def blk_body(blk, _):
            acc16 = jnp.zeros((LANES,), jnp.float32)
            for rr in range(LANES):
                k = blk * LANES + rr
                parts = []
                for j in range(FEAT // LANES):
                    u = urows_v[k, 0, pl.ds(j * LANES, LANES)]
                    i = irows_v[k, 0, pl.ds(j * LANES, LANES)]
                    parts.append(u * i)
                s = (parts[0] + parts[1]) + (parts[2] + parts[3])
                tot = jnp.sum(s)
                acc16 = jnp.where(lane == rr, tot, acc16)
            out_v[pl.ds(lo + blk * LANES, LANES)] = acc16
            return ()

        lax.fori_loop(0, CHUNK // LANES, blk_body, ())
        return ()

    lax.fori_loop(0, NCHUNK, chunk_body, ())

    pltpu.sync_copy(out_v, out_hbm.at[pl.ds(base, ROWS_PER_W)])


@jax.jit
def kernel(user, item, users_emb, items_emb):
    mesh = plsc.VectorSubcoreMesh(core_axis_name="c", subcore_axis_name="s")
    k = pl.kernel(
        _body,
        out_type=jax.ShapeDtypeStruct((BATCH_N,), jnp.float32),
        mesh=mesh,
        scratch_types=[
            pltpu.VMEM((ROWS_PER_W,), jnp.int32),
            pltpu.VMEM((ROWS_PER_W,), jnp.int32),
            pltpu.VMEM((CHUNK, 1, FEAT), jnp.float32),
            pltpu.VMEM((CHUNK, 1, FEAT), jnp.float32),
            pltpu.VMEM((ROWS_PER_W,), jnp.float32),
            pltpu.SemaphoreType.DMA,
        ],
        compiler_params=pltpu.CompilerParams(needs_layout_passes=False),
    )
    return k(user.astype(jnp.int32), item.astype(jnp.int32),
             users_emb, items_emb)
